# baseline (speedup 1.0000x reference)
import functools

import jax
import jax.numpy as jnp
from jax import lax
from jax.experimental import pallas as pl
from jax.experimental.pallas import tpu as pltpu
from jax.experimental.pallas import tpu_sc as plsc

F32 = jnp.float32

_NW = 32
_NS = 16
_CH = 128


@functools.lru_cache(maxsize=None)
def _sc_deg_build(NP, T):
    mesh = plsc.VectorSubcoreMesh(core_axis_name="c", subcore_axis_name="s")
    rpt = NP // _NS

    @functools.partial(
        pl.kernel,
        out_type=jax.ShapeDtypeStruct((2, NP, 128), F32),
        mesh=mesh,
        scratch_types=[
            pltpu.VMEM((T, _CH), jnp.int32),
            pltpu.VMEM((_CH, 128), F32),
            pltpu.VMEM_SHARED((NP, 128), F32),
            pltpu.SemaphoreType.DMA,
        ],
    )
    def deg_fn(cols_hbm, ones_hbm, zeros_hbm, out_hbm, colv, buf, acc, sem):
        c = lax.axis_index("c")
        s = lax.axis_index("s")
        wid = c * _NS + s
        pltpu.sync_copy(zeros_hbm, acc.at[pl.ds(s * rpt, rpt)])
        pltpu.sync_copy(cols_hbm.at[wid], colv)
        pltpu.sync_copy(ones_hbm, buf)
        plsc.subcore_barrier()

        def body(j, carry):
            pltpu.sync_copy(buf, acc.at[colv.at[j]], add=True)
            return carry

        lax.fori_loop(0, T, body, 0)
        plsc.subcore_barrier()
        pltpu.sync_copy(acc.at[pl.ds(s * rpt, rpt)],
                        out_hbm.at[c, pl.ds(s * rpt, rpt)])

    return deg_fn


@functools.lru_cache(maxsize=None)
def _sc_scatter_build(NP, W, T):
    mesh = plsc.VectorSubcoreMesh(core_axis_name="c", subcore_axis_name="s")
    rpt = NP // _NS

    NB = 2
    WIN = 8

    @functools.partial(
        pl.kernel,
        out_type=jax.ShapeDtypeStruct((2, NP, W), F32),
        mesh=mesh,
        scratch_types=[
            pltpu.VMEM((T, _CH), jnp.int32),
            pltpu.VMEM((2, WIN, _CH), jnp.int32),
            pltpu.VMEM((_CH, W), F32),
            pltpu.VMEM((_CH, W), F32),
            pltpu.VMEM_SHARED((NP, W), F32),
            pltpu.SemaphoreType.DMA,
            pltpu.SemaphoreType.DMA,
            pltpu.SemaphoreType.DMA,
        ],
    )
    def sc_fn(msg_hbm, rows_hbm, cols_hbm, zeros_hbm, out_hbm,
              rowv, colw, b0, b1, acc, g0, g1, isem):
        bufs = (b0, b1)
        gsem = (g0, g1)
        c = lax.axis_index("c")
        s = lax.axis_index("s")
        wid = c * _NS + s
        pltpu.sync_copy(zeros_hbm, acc.at[pl.ds(s * rpt, rpt)])
        pltpu.sync_copy(rows_hbm.at[wid], rowv)
        pltpu.sync_copy(cols_hbm.at[wid, pl.ds(0, WIN)], colw.at[0])
        plsc.subcore_barrier()

        NWIN = T // WIN
        pltpu.async_copy(cols_hbm.at[wid, pl.ds(WIN, WIN)], colw.at[1], isem)
        for p in range(NB):
            pltpu.async_copy(msg_hbm.at[rowv.at[p]], bufs[p], gsem[p])

        def do_window(w, slot):
            for p in range(WIN):
                j = w * WIN + p
                pltpu.make_async_copy(msg_hbm.at[rowv.at[j]], bufs[p % NB],
                                      gsem[p % NB]).wait()
                pltpu.sync_copy(bufs[p % NB], acc.at[colw.at[slot, p]],
                                add=True)

                @pl.when(j + NB < T)
                def _():
                    pltpu.async_copy(msg_hbm.at[rowv.at[j + NB]],
                                     bufs[p % NB], gsem[p % NB])

        def outer(k, carry):
            w0 = 2 * k

            @pl.when(k > 0)
            def _():
                pltpu.make_async_copy(cols_hbm.at[wid, pl.ds(0, WIN)],
                                      colw.at[0], isem).wait()

            do_window(w0, 0)

            @pl.when(w0 + 2 < NWIN)
            def _():
                pltpu.async_copy(cols_hbm.at[wid, pl.ds((w0 + 2) * WIN, WIN)],
                                 colw.at[0], isem)

            pltpu.make_async_copy(cols_hbm.at[wid, pl.ds(0, WIN)],
                                  colw.at[1], isem).wait()
            do_window(w0 + 1, 1)

            @pl.when(w0 + 3 < NWIN)
            def _():
                pltpu.async_copy(cols_hbm.at[wid, pl.ds((w0 + 3) * WIN, WIN)],
                                 colw.at[1], isem)
            return carry

        lax.fori_loop(0, NWIN // 2, outer, 0)
        plsc.subcore_barrier()
        pltpu.sync_copy(acc.at[pl.ds(s * rpt, rpt)],
                        out_hbm.at[c, pl.ds(s * rpt, rpt)])

    return sc_fn


def _sc_scatter_call(msg, rows_p, cols_p, zeros, NP, W, T):
    return _sc_scatter_build(NP, W, T)(msg, rows_p, cols_p, zeros)


def _sc_deg_call(cols_p, ones_ch, zeros, NP, T):
    return _sc_deg_build(NP, T)(cols_p, ones_ch, zeros)


def _dot(a, b):
    return jnp.dot(a, b, preferred_element_type=F32)


def _precompute_call(N, B, F, S, degparts, x, batch2, batchT, iv,
                     w1a1, b11, w21, b21, w1a2, b12, w1c2):
    def body(dp_ref, x_ref, b_ref, bT_ref, iv_ref,
             w1a1_ref, b11_ref, w21_ref, b21_ref,
             w1a2_ref, b12_ref, w1c2_ref,
             A_out, msg1_out, P2_out, U1_out, OH_out, OHT_out, cntF_out,
             deg_out):
        xv = x_ref[...]
        A = _dot(xv, w1a1_ref[...]) + b11_ref[...]
        A_out[...] = A
        msg1_out[...] = _dot(jnp.maximum(A, 0.0), w21_ref[...]) + b21_ref[...]
        P2_out[...] = _dot(xv, w1a2_ref[...]) + b12_ref[...]
        U1_out[...] = _dot(iv_ref[...], w1c2_ref[...])
        OH = (b_ref[...] == lax.broadcasted_iota(jnp.int32, (N, B), 1)).astype(F32)
        OH_out[...] = OH
        OHT_out[...] = (bT_ref[...] == lax.broadcasted_iota(jnp.int32, (B, N), 0)).astype(F32)
        cnt = jnp.sum(OH, axis=0)[:, None]
        cntF_out[...] = jnp.maximum(cnt * float(F), 1.0)
        deg_out[...] = jnp.maximum(dp_ref[0, :N, 0:1] + dp_ref[1, :N, 0:1], 1.0)

    return pl.pallas_call(
        body,
        out_shape=[
            jax.ShapeDtypeStruct((N, F), F32),
            jax.ShapeDtypeStruct((N, F), F32),
            jax.ShapeDtypeStruct((N, F), F32),
            jax.ShapeDtypeStruct((S * B, F), F32),
            jax.ShapeDtypeStruct((N, B), F32),
            jax.ShapeDtypeStruct((B, N), F32),
            jax.ShapeDtypeStruct((B, 1), F32),
            jax.ShapeDtypeStruct((N, 1), F32),
        ],
    )(degparts, x, batch2, batchT, iv, w1a1, b11, w21, b21, w1a2, b12, w1c2)


def _step_math(N, F, Ssum, deg, xv, P2, OH, OHT, cntF, U1s,
               w1b2, w22, b22, lnw, lnb, gw1, gb1, gw2, gb2,
               hw1, hb1, hw2, hb2):
    agg = Ssum[:, :F] / deg
    h = jnp.maximum(P2 + _dot(agg, w1b2) + _dot(OH, U1s), 0.0)
    x1 = _dot(h, w22) + b22 + xv
    rs = jnp.sum(x1, axis=1, keepdims=True)
    mean = _dot(OHT, rs) / cntF
    xc = x1 - _dot(OH, mean)
    rss = jnp.sum(xc * xc, axis=1, keepdims=True)
    var = _dot(OHT, rss) / cntF
    xn = xc / _dot(OH, jnp.sqrt(var + 1e-5)) * lnw + lnb
    g = _dot(jnp.maximum(_dot(xn, gw1) + gb1, 0.0), gw2) + gb2
    m = jnp.max(jnp.where(OH > 0.0, g, -jnp.inf), axis=0)[:, None]
    m = jnp.where(jnp.isfinite(m), m, 0.0)
    e = jnp.exp(g - _dot(OH, m))
    ssum = _dot(OHT, e)
    gate = e / (_dot(OH, ssum) + 1e-16)
    hist = _dot(OHT, gate * xn)
    hv = _dot(jnp.maximum(_dot(hist, hw1) + hb1, 0.0), hw2) + hb2
    return gate, hv, hist


def _step_call(N, NP, B, F, part, xv, P2, OH, OHT, cntF, U1s, deg,
               *weights):
    def body(p_ref, x_ref, P2_ref, OH_ref, OHT_ref, cntF_ref,
             U1s_ref, deg_ref, *rest):
        wrefs = rest[:13]
        gate_out, hv_out, hist_out = rest[13:]
        Ssum = p_ref[0, :N, :] + p_ref[1, :N, :]
        gate, hv, hist = _step_math(
            N, F, Ssum, deg_ref[...], x_ref[...], P2_ref[...],
            OH_ref[...], OHT_ref[...], cntF_ref[...], U1s_ref[...],
            *[w[...] for w in wrefs])
        gate_out[...] = gate
        hv_out[...] = hv
        hist_out[...] = hist

    out_shape = [
        jax.ShapeDtypeStruct((N, 1), F32),
        jax.ShapeDtypeStruct((B, F), F32),
        jax.ShapeDtypeStruct((B, F), F32),
    ]
    args = (part, xv, P2, OH, OHT, cntF, U1s, deg) + weights
    return pl.pallas_call(body, out_shape=out_shape)(*args)


def _msg_call(N, B, F, A, OH, hist, w1b1, w21, b21):
    def body(A_ref, OH_ref, hist_ref, w1b1_ref, w21_ref, b21_ref, msg_out):
        pre = A_ref[...] + _dot(OH_ref[...], _dot(hist_ref[...], w1b1_ref[...]))
        msg_out[...] = _dot(jnp.maximum(pre, 0.0), w21_ref[...]) + b21_ref[...]

    return pl.pallas_call(
        body, out_shape=jax.ShapeDtypeStruct((N, F), F32),
    )(A, OH, hist, w1b1, w21, b21)


def kernel(x, edge_attr, instr_vectors, nm1_w1, nm1_b1, nm1_w2, nm1_b2,
           nm2_w1, nm2_b1, nm2_w2, nm2_b2, ln_w, ln_b,
           gm_w1, gm_b1, gm_w2, gm_b2, hm_w1, hm_b1, hm_w2, hm_b2,
           edge_index, batch):
    N, F = x.shape
    E = edge_index.shape[1]
    S, B, FI = instr_vectors.shape

    NP = (N // 1024 + 1) * 1024
    T = -(-E // (_NW * _CH))
    T = -(-T // 16) * 16
    EP = _NW * T * _CH

    row = edge_index[0]
    col = edge_index[1]
    pad = EP - E
    rows_p = jnp.concatenate([row, jnp.zeros((pad,), jnp.int32)]).reshape(_NW, T, _CH)
    cols_p = jnp.concatenate([col, jnp.full((pad,), N, jnp.int32)]).reshape(_NW, T, _CH)

    ones_ch = jnp.ones((_CH, 128), F32)
    zeros_n = jnp.zeros((NP // _NS, F), F32)

    degparts = _sc_deg_call(cols_p, ones_ch, zeros_n, NP, T)

    A, msg, P2, U1all, OH, OHT, cntF, deg = _precompute_call(
        N, B, F, S, degparts, x, batch[:, None], batch[None, :],
        instr_vectors.reshape(S * B, FI),
        nm1_w1[:F], nm1_b1[None, :], nm1_w2, nm1_b2[None, :],
        nm2_w1[:F], nm2_b1[None, :], nm2_w1[2 * F:],
    )

    step_weights = (
        nm2_w1[F:2 * F], nm2_w2, nm2_b2[None, :],
        ln_w[None, :], ln_b[None, :],
        gm_w1, gm_b1[None, :], gm_w2, gm_b2[None, :],
        hm_w1, hm_b1[None, :], hm_w2, hm_b2[None, :],
    )

    gates = []
    hvs = []
    for s in range(S):
        part = _sc_scatter_call(msg, rows_p, cols_p, zeros_n, NP, F, T)
        gate, hv_s, hist = _step_call(
            N, NP, B, F, part, x, P2, OH, OHT, cntF,
            U1all[s * B:(s + 1) * B], deg, *step_weights)
        gates.append(gate)
        hvs.append(hv_s)
        if s + 1 < S:
            msg = _msg_call(N, B, F, A, OH, hist, nm1_w1[F:], nm1_w2,
                            nm1_b2[None, :])

    execution_bitmap = jnp.concatenate(gates, axis=1)
    hv = jnp.stack(hvs, axis=0)
    return (x, execution_bitmap, hv)


# speedup vs baseline: 2.8526x; 2.8526x over previous
import functools

import jax
import jax.numpy as jnp
from jax import lax
from jax.experimental import pallas as pl
from jax.experimental.pallas import tpu as pltpu
from jax.experimental.pallas import tpu_sc as plsc

F32 = jnp.float32

_NW = 32
_NS = 16
_CH = 128


@functools.lru_cache(maxsize=None)
def _sc_deg_build(NP, T):
    mesh = plsc.VectorSubcoreMesh(core_axis_name="c", subcore_axis_name="s")
    rpt = NP // _NS

    @functools.partial(
        pl.kernel,
        out_type=jax.ShapeDtypeStruct((2, NP, 128), F32),
        mesh=mesh,
        scratch_types=[
            pltpu.VMEM((T, _CH), jnp.int32),
            pltpu.VMEM((_CH, 128), F32),
            pltpu.VMEM_SHARED((NP, 128), F32),
            pltpu.SemaphoreType.DMA,
        ],
    )
    def deg_fn(cols_hbm, ones_hbm, zeros_hbm, out_hbm, colv, buf, acc, sem):
        c = lax.axis_index("c")
        s = lax.axis_index("s")
        wid = c * _NS + s
        pltpu.sync_copy(zeros_hbm, acc.at[pl.ds(s * rpt, rpt)])
        pltpu.sync_copy(cols_hbm.at[wid], colv)
        pltpu.sync_copy(ones_hbm, buf)
        plsc.subcore_barrier()

        def body(j, carry):
            pltpu.sync_copy(buf, acc.at[colv.at[j]], add=True)
            return carry

        lax.fori_loop(0, T, body, 0)
        plsc.subcore_barrier()
        pltpu.sync_copy(acc.at[pl.ds(s * rpt, rpt)],
                        out_hbm.at[c, pl.ds(s * rpt, rpt)])

    return deg_fn


@functools.lru_cache(maxsize=None)
def _sc_scatter_build(NP, W, T):
    mesh = plsc.VectorSubcoreMesh(core_axis_name="c", subcore_axis_name="s")
    rpt = NP // _NS

    NB = 2
    WIN = 8

    @functools.partial(
        pl.kernel,
        out_type=jax.ShapeDtypeStruct((2, NP, W), F32),
        mesh=mesh,
        scratch_types=[
            pltpu.VMEM((T, _CH), jnp.int32),
            pltpu.VMEM((2, WIN, _CH), jnp.int32),
            pltpu.VMEM((_CH, W), F32),
            pltpu.VMEM((_CH, W), F32),
            pltpu.VMEM_SHARED((NP, W), F32),
            pltpu.SemaphoreType.DMA,
            pltpu.SemaphoreType.DMA,
            pltpu.SemaphoreType.DMA,
        ],
    )
    def sc_fn(msg_hbm, rows_hbm, cols_hbm, zeros_hbm, out_hbm,
              rowv, colw, b0, b1, acc, g0, g1, isem):
        bufs = (b0, b1)
        gsem = (g0, g1)
        c = lax.axis_index("c")
        s = lax.axis_index("s")
        wid = c * _NS + s
        pltpu.sync_copy(zeros_hbm, acc.at[pl.ds(s * rpt, rpt)])
        pltpu.sync_copy(rows_hbm.at[wid], rowv)
        pltpu.sync_copy(cols_hbm.at[wid, pl.ds(0, WIN)], colw.at[0])
        plsc.subcore_barrier()

        NWIN = T // WIN
        pltpu.async_copy(cols_hbm.at[wid, pl.ds(WIN, WIN)], colw.at[1], isem)
        for p in range(NB):
            pltpu.async_copy(msg_hbm.at[rowv.at[p]], bufs[p], gsem[p])

        def do_window(w, slot):
            for p in range(WIN):
                j = w * WIN + p
                pltpu.make_async_copy(msg_hbm.at[rowv.at[j]], bufs[p % NB],
                                      gsem[p % NB]).wait()
                pltpu.sync_copy(bufs[p % NB], acc.at[colw.at[slot, p]],
                                add=True)

                @pl.when(j + NB < T)
                def _():
                    pltpu.async_copy(msg_hbm.at[rowv.at[j + NB]],
                                     bufs[p % NB], gsem[p % NB])

        def outer(k, carry):
            w0 = 2 * k

            @pl.when(k > 0)
            def _():
                pltpu.make_async_copy(cols_hbm.at[wid, pl.ds(0, WIN)],
                                      colw.at[0], isem).wait()

            do_window(w0, 0)

            @pl.when(w0 + 2 < NWIN)
            def _():
                pltpu.async_copy(cols_hbm.at[wid, pl.ds((w0 + 2) * WIN, WIN)],
                                 colw.at[0], isem)

            pltpu.make_async_copy(cols_hbm.at[wid, pl.ds(0, WIN)],
                                  colw.at[1], isem).wait()
            do_window(w0 + 1, 1)

            @pl.when(w0 + 3 < NWIN)
            def _():
                pltpu.async_copy(cols_hbm.at[wid, pl.ds((w0 + 3) * WIN, WIN)],
                                 colw.at[1], isem)
            return carry

        lax.fori_loop(0, NWIN // 2, outer, 0)
        plsc.subcore_barrier()
        pltpu.sync_copy(acc.at[pl.ds(s * rpt, rpt)],
                        out_hbm.at[c, pl.ds(s * rpt, rpt)])

    return sc_fn


def _sc_scatter_call(msg, rows_p, cols_p, zeros, NP, W, T):
    return _sc_scatter_build(NP, W, T)(msg, rows_p, cols_p, zeros)


def _sc_deg_call(cols_p, ones_ch, zeros, NP, T):
    return _sc_deg_build(NP, T)(cols_p, ones_ch, zeros)


def _dot(a, b):
    return jnp.dot(a, b, preferred_element_type=F32)


def _precompute_call(N, B, F, S, degparts, x, batch2, batchT, iv,
                     w1a1, b11, w21, b21, w1a2, b12, w1c2):
    def body(dp_ref, x_ref, b_ref, bT_ref, iv_ref,
             w1a1_ref, b11_ref, w21_ref, b21_ref,
             w1a2_ref, b12_ref, w1c2_ref,
             A_out, msg1_out, P2_out, U1_out, OH_out, OHT_out, cntF_out,
             deg_out):
        xv = x_ref[...]
        A = _dot(xv, w1a1_ref[...]) + b11_ref[...]
        A_out[...] = A
        msg1_out[...] = _dot(jnp.maximum(A, 0.0), w21_ref[...]) + b21_ref[...]
        P2_out[...] = _dot(xv, w1a2_ref[...]) + b12_ref[...]
        U1_out[...] = _dot(iv_ref[...], w1c2_ref[...])
        OH = (b_ref[...] == lax.broadcasted_iota(jnp.int32, (N, B), 1)).astype(F32)
        OH_out[...] = OH
        OHT_out[...] = (bT_ref[...] == lax.broadcasted_iota(jnp.int32, (B, N), 0)).astype(F32)
        cnt = jnp.sum(OH, axis=0)[:, None]
        cntF_out[...] = jnp.maximum(cnt * float(F), 1.0)
        deg_out[...] = jnp.maximum(dp_ref[0, :N, 0:1] + dp_ref[1, :N, 0:1], 1.0)

    return pl.pallas_call(
        body,
        out_shape=[
            jax.ShapeDtypeStruct((N, F), F32),
            jax.ShapeDtypeStruct((N, F), F32),
            jax.ShapeDtypeStruct((N, F), F32),
            jax.ShapeDtypeStruct((S * B, F), F32),
            jax.ShapeDtypeStruct((N, B), F32),
            jax.ShapeDtypeStruct((B, N), F32),
            jax.ShapeDtypeStruct((B, 1), F32),
            jax.ShapeDtypeStruct((N, 1), F32),
        ],
    )(degparts, x, batch2, batchT, iv, w1a1, b11, w21, b21, w1a2, b12, w1c2)


def _step_math(N, F, Ssum, deg, xv, P2, OH, OHT, cntF, U1s,
               w1b2, w22, b22, lnw, lnb, gw1, gb1, gw2, gb2,
               hw1, hb1, hw2, hb2):
    agg = Ssum[:, :F] / deg
    h = jnp.maximum(P2 + _dot(agg, w1b2) + _dot(OH, U1s), 0.0)
    x1 = _dot(h, w22) + b22 + xv
    rs = jnp.sum(x1, axis=1, keepdims=True)
    mean = _dot(OHT, rs) / cntF
    xc = x1 - _dot(OH, mean)
    rss = jnp.sum(xc * xc, axis=1, keepdims=True)
    var = _dot(OHT, rss) / cntF
    xn = xc / _dot(OH, jnp.sqrt(var + 1e-5)) * lnw + lnb
    g = _dot(jnp.maximum(_dot(xn, gw1) + gb1, 0.0), gw2) + gb2
    m = jnp.max(jnp.where(OH > 0.0, g, -jnp.inf), axis=0)[:, None]
    m = jnp.where(jnp.isfinite(m), m, 0.0)
    e = jnp.exp(g - _dot(OH, m))
    ssum = _dot(OHT, e)
    gate = e / (_dot(OH, ssum) + 1e-16)
    hist = _dot(OHT, gate * xn)
    hv = _dot(jnp.maximum(_dot(hist, hw1) + hb1, 0.0), hw2) + hb2
    return gate, hv, hist


def _step_call(N, NP, B, F, part, xv, P2, OH, OHT, cntF, U1s, deg,
               *weights):
    def body(p_ref, x_ref, P2_ref, OH_ref, OHT_ref, cntF_ref,
             U1s_ref, deg_ref, *rest):
        wrefs = rest[:13]
        gate_out, hv_out, hist_out = rest[13:]
        Ssum = p_ref[0, :N, :] + p_ref[1, :N, :]
        gate, hv, hist = _step_math(
            N, F, Ssum, deg_ref[...], x_ref[...], P2_ref[...],
            OH_ref[...], OHT_ref[...], cntF_ref[...], U1s_ref[...],
            *[w[...] for w in wrefs])
        gate_out[...] = gate
        hv_out[...] = hv
        hist_out[...] = hist

    out_shape = [
        jax.ShapeDtypeStruct((N, 1), F32),
        jax.ShapeDtypeStruct((B, F), F32),
        jax.ShapeDtypeStruct((B, F), F32),
    ]
    args = (part, xv, P2, OH, OHT, cntF, U1s, deg) + weights
    return pl.pallas_call(body, out_shape=out_shape)(*args)


def _msg_call(N, B, F, A, OH, hist, w1b1, w21, b21):
    def body(A_ref, OH_ref, hist_ref, w1b1_ref, w21_ref, b21_ref, msg_out):
        pre = A_ref[...] + _dot(OH_ref[...], _dot(hist_ref[...], w1b1_ref[...]))
        msg_out[...] = _dot(jnp.maximum(pre, 0.0), w21_ref[...]) + b21_ref[...]

    return pl.pallas_call(
        body, out_shape=jax.ShapeDtypeStruct((N, F), F32),
    )(A, OH, hist, w1b1, w21, b21)


def kernel(x, edge_attr, instr_vectors, nm1_w1, nm1_b1, nm1_w2, nm1_b2,
           nm2_w1, nm2_b1, nm2_w2, nm2_b2, ln_w, ln_b,
           gm_w1, gm_b1, gm_w2, gm_b2, hm_w1, hm_b1, hm_w2, hm_b2,
           edge_index, batch):
    N, F = x.shape
    E = edge_index.shape[1]
    S, B, FI = instr_vectors.shape

    NP = (N // 1024 + 1) * 1024
    T = -(-E // (_NW * _CH))
    T = -(-T // 16) * 16
    EP = _NW * T * _CH

    row = edge_index[0]
    col = edge_index[1]
    pad = EP - E
    pad_i = jnp.arange(pad, dtype=jnp.int32)
    rows_p = jnp.concatenate([row, pad_i % N]).reshape(_NW, T, _CH)
    cols_p = jnp.concatenate([col, N + pad_i % (NP - N)]).reshape(_NW, T, _CH)

    ones_ch = jnp.ones((_CH, 128), F32)
    zeros_n = jnp.zeros((NP // _NS, F), F32)

    degparts = _sc_deg_call(cols_p, ones_ch, zeros_n, NP, T)

    A, msg, P2, U1all, OH, OHT, cntF, deg = _precompute_call(
        N, B, F, S, degparts, x, batch[:, None], batch[None, :],
        instr_vectors.reshape(S * B, FI),
        nm1_w1[:F], nm1_b1[None, :], nm1_w2, nm1_b2[None, :],
        nm2_w1[:F], nm2_b1[None, :], nm2_w1[2 * F:],
    )

    step_weights = (
        nm2_w1[F:2 * F], nm2_w2, nm2_b2[None, :],
        ln_w[None, :], ln_b[None, :],
        gm_w1, gm_b1[None, :], gm_w2, gm_b2[None, :],
        hm_w1, hm_b1[None, :], hm_w2, hm_b2[None, :],
    )

    gates = []
    hvs = []
    for s in range(S):
        part = _sc_scatter_call(msg, rows_p, cols_p, zeros_n, NP, F, T)
        gate, hv_s, hist = _step_call(
            N, NP, B, F, part, x, P2, OH, OHT, cntF,
            U1all[s * B:(s + 1) * B], deg, *step_weights)
        gates.append(gate)
        hvs.append(hv_s)
        if s + 1 < S:
            msg = _msg_call(N, B, F, A, OH, hist, nm1_w1[F:], nm1_w2,
                            nm1_b2[None, :])

    execution_bitmap = jnp.concatenate(gates, axis=1)
    hv = jnp.stack(hvs, axis=0)
    return (x, execution_bitmap, hv)


# speedup vs baseline: 3.0812x; 1.0801x over previous
import functools

import jax
import jax.numpy as jnp
from jax import lax
from jax.experimental import pallas as pl
from jax.experimental.pallas import tpu as pltpu
from jax.experimental.pallas import tpu_sc as plsc

F32 = jnp.float32

_NW = 32
_NS = 16
_CH = 128


def _acc_writeout(N, NP, acc, out_hbm, c, s):
    rpt = NP // _NS
    full = N // rpt
    rem = N - full * rpt

    @pl.when(s < full)
    def _():
        pltpu.sync_copy(acc.at[pl.ds(s * rpt, rpt)],
                        out_hbm.at[c, pl.ds(s * rpt, rpt)])

    if rem:
        @pl.when(s == full)
        def _():
            pltpu.sync_copy(acc.at[pl.ds(full * rpt, rem)],
                            out_hbm.at[c, pl.ds(full * rpt, rem)])


@functools.lru_cache(maxsize=None)
def _sc_deg_build(N, NP, T):
    mesh = plsc.VectorSubcoreMesh(core_axis_name="c", subcore_axis_name="s")
    rpt = NP // _NS

    @functools.partial(
        pl.kernel,
        out_type=jax.ShapeDtypeStruct((2, N, 128), F32),
        mesh=mesh,
        scratch_types=[
            pltpu.VMEM((T, _CH), jnp.int32),
            pltpu.VMEM((_CH, 128), F32),
            pltpu.VMEM_SHARED((NP, 128), F32),
            pltpu.SemaphoreType.DMA,
        ],
    )
    def deg_fn(cols_hbm, ones_hbm, zeros_hbm, out_hbm, colv, buf, acc, sem):
        c = lax.axis_index("c")
        s = lax.axis_index("s")
        wid = c * _NS + s
        pltpu.sync_copy(zeros_hbm, acc.at[pl.ds(s * rpt, rpt)])
        pltpu.sync_copy(cols_hbm.at[wid], colv)
        pltpu.sync_copy(ones_hbm, buf)
        plsc.subcore_barrier()

        def body(j, carry):
            pltpu.sync_copy(buf, acc.at[colv.at[j]], add=True)
            return carry

        lax.fori_loop(0, T, body, 0)
        plsc.subcore_barrier()
        _acc_writeout(N, NP, acc, out_hbm, c, s)

    return deg_fn


@functools.lru_cache(maxsize=None)
def _sc_scatter_build(N, NP, W, T):
    mesh = plsc.VectorSubcoreMesh(core_axis_name="c", subcore_axis_name="s")
    rpt = NP // _NS

    NB = 2
    WIN = 8

    @functools.partial(
        pl.kernel,
        out_type=jax.ShapeDtypeStruct((2, N, W), F32),
        mesh=mesh,
        scratch_types=[
            pltpu.VMEM((T, _CH), jnp.int32),
            pltpu.VMEM((2, WIN, _CH), jnp.int32),
            pltpu.VMEM((_CH, W), F32),
            pltpu.VMEM((_CH, W), F32),
            pltpu.VMEM_SHARED((NP, W), F32),
            pltpu.SemaphoreType.DMA,
            pltpu.SemaphoreType.DMA,
            pltpu.SemaphoreType.DMA,
        ],
    )
    def sc_fn(msg_hbm, rows_hbm, cols_hbm, zeros_hbm, out_hbm,
              rowv, colw, b0, b1, acc, g0, g1, isem):
        bufs = (b0, b1)
        gsem = (g0, g1)
        c = lax.axis_index("c")
        s = lax.axis_index("s")
        wid = c * _NS + s
        pltpu.sync_copy(zeros_hbm, acc.at[pl.ds(s * rpt, rpt)])
        pltpu.sync_copy(rows_hbm.at[wid], rowv)
        pltpu.sync_copy(cols_hbm.at[wid, pl.ds(0, WIN)], colw.at[0])
        plsc.subcore_barrier()

        NWIN = T // WIN
        pltpu.async_copy(cols_hbm.at[wid, pl.ds(WIN, WIN)], colw.at[1], isem)
        for p in range(NB):
            pltpu.async_copy(msg_hbm.at[rowv.at[p]], bufs[p], gsem[p])

        def do_window(w, slot):
            for p in range(WIN):
                j = w * WIN + p
                pltpu.make_async_copy(msg_hbm.at[rowv.at[j]], bufs[p % NB],
                                      gsem[p % NB]).wait()
                pltpu.sync_copy(bufs[p % NB], acc.at[colw.at[slot, p]],
                                add=True)

                @pl.when(j + NB < T)
                def _():
                    pltpu.async_copy(msg_hbm.at[rowv.at[j + NB]],
                                     bufs[p % NB], gsem[p % NB])

        def outer(k, carry):
            w0 = 2 * k

            @pl.when(k > 0)
            def _():
                pltpu.make_async_copy(cols_hbm.at[wid, pl.ds(0, WIN)],
                                      colw.at[0], isem).wait()

            do_window(w0, 0)

            @pl.when(w0 + 2 < NWIN)
            def _():
                pltpu.async_copy(cols_hbm.at[wid, pl.ds((w0 + 2) * WIN, WIN)],
                                 colw.at[0], isem)

            pltpu.make_async_copy(cols_hbm.at[wid, pl.ds(0, WIN)],
                                  colw.at[1], isem).wait()
            do_window(w0 + 1, 1)

            @pl.when(w0 + 3 < NWIN)
            def _():
                pltpu.async_copy(cols_hbm.at[wid, pl.ds((w0 + 3) * WIN, WIN)],
                                 colw.at[1], isem)
            return carry

        lax.fori_loop(0, NWIN // 2, outer, 0)
        plsc.subcore_barrier()
        _acc_writeout(N, NP, acc, out_hbm, c, s)

    return sc_fn


def _sc_scatter_call(msg, rows_p, cols_p, zeros, N, NP, W, T):
    return _sc_scatter_build(N, NP, W, T)(msg, rows_p, cols_p, zeros)


def _sc_deg_call(cols_p, ones_ch, zeros, N, NP, T):
    return _sc_deg_build(N, NP, T)(cols_p, ones_ch, zeros)


def _dot(a, b):
    return jnp.dot(a, b, preferred_element_type=F32)


def _dotT(a, b):
    return lax.dot_general(a, b, (((0,), (0,)), ((), ())),
                           preferred_element_type=F32)


def _precompute_call(N, B, F, S, x, batch2, iv,
                     w1a1, b11, w21, b21, w1c2):
    def body(x_ref, b_ref, iv_ref,
             w1a1_ref, b11_ref, w21_ref, b21_ref, w1c2_ref,
             msg1_out, U1_out, OH_out, cntF_out):
        xv = x_ref[...]
        A = _dot(xv, w1a1_ref[...]) + b11_ref[...]
        msg1_out[...] = _dot(jnp.maximum(A, 0.0), w21_ref[...]) + b21_ref[...]
        U1_out[...] = _dot(iv_ref[...], w1c2_ref[...])
        OH = (b_ref[...] == lax.broadcasted_iota(jnp.int32, (N, B), 1)).astype(F32)
        OH_out[...] = OH
        cnt = jnp.sum(OH, axis=0)[:, None]
        cntF_out[...] = jnp.maximum(cnt * float(F), 1.0)

    return pl.pallas_call(
        body,
        out_shape=[
            jax.ShapeDtypeStruct((N, F), F32),
            jax.ShapeDtypeStruct((S * B, F), F32),
            jax.ShapeDtypeStruct((N, B), F32),
            jax.ShapeDtypeStruct((B, 1), F32),
        ],
    )(x, batch2, iv, w1a1, b11, w21, b21, w1c2)


def _deg_reduce_call(N, NP, degparts):
    def body(dp_ref, deg_out):
        deg_out[...] = jnp.maximum(dp_ref[0, :, 0:1] + dp_ref[1, :, 0:1],
                                   1.0)

    return pl.pallas_call(
        body, out_shape=jax.ShapeDtypeStruct((N, 1), F32),
    )(degparts)


def _step_math(N, F, Ssum, deg, xv, OH, cntF, U1s,
               w1a2, b12, w1b2, w22, b22, lnw, lnb, gw1, gb1, gw2, gb2,
               hw1, hb1, hw2, hb2, w1a1, b11, w1b1, w21, b21):
    agg = Ssum[:, :F] / deg
    P2 = _dot(xv, w1a2) + b12
    h = jnp.maximum(P2 + _dot(agg, w1b2) + _dot(OH, U1s), 0.0)
    x1 = _dot(h, w22) + b22 + xv
    rs = jnp.sum(x1, axis=1, keepdims=True)
    mean = _dotT(OH, rs) / cntF
    xc = x1 - _dot(OH, mean)
    rss = jnp.sum(xc * xc, axis=1, keepdims=True)
    var = _dotT(OH, rss) / cntF
    xn = xc / _dot(OH, jnp.sqrt(var + 1e-5)) * lnw + lnb
    g = _dot(jnp.maximum(_dot(xn, gw1) + gb1, 0.0), gw2) + gb2
    m = jnp.max(jnp.where(OH > 0.0, g, -jnp.inf), axis=0)[:, None]
    m = jnp.where(jnp.isfinite(m), m, 0.0)
    e = jnp.exp(g - _dot(OH, m))
    ssum = _dotT(OH, e)
    gate = e / (_dot(OH, ssum) + 1e-16)
    hist = _dotT(OH, gate * xn)
    hv = _dot(jnp.maximum(_dot(hist, hw1) + hb1, 0.0), hw2) + hb2
    pre = _dot(xv, w1a1) + b11 + _dot(OH, _dot(hist, w1b1))
    msg = _dot(jnp.maximum(pre, 0.0), w21) + b21
    return gate, hv, msg


def _step_call(N, NP, B, F, part, xv, OH, cntF, U1s, deg,
               *weights):
    def body(p_ref, x_ref, OH_ref, cntF_ref,
             U1s_ref, deg_ref, *rest):
        wrefs = rest[:20]
        gate_out, hv_out, msg_out = rest[20:]
        Ssum = p_ref[0] + p_ref[1]
        gate, hv, msg = _step_math(
            N, F, Ssum, deg_ref[...], x_ref[...],
            OH_ref[...], cntF_ref[...], U1s_ref[...],
            *[w[...] for w in wrefs])
        gate_out[...] = gate
        hv_out[...] = hv
        msg_out[...] = msg

    out_shape = [
        jax.ShapeDtypeStruct((N, 1), F32),
        jax.ShapeDtypeStruct((B, F), F32),
        jax.ShapeDtypeStruct((N, F), F32),
    ]
    args = (part, xv, OH, cntF, U1s, deg) + weights
    return pl.pallas_call(body, out_shape=out_shape)(*args)


def kernel(x, edge_attr, instr_vectors, nm1_w1, nm1_b1, nm1_w2, nm1_b2,
           nm2_w1, nm2_b1, nm2_w2, nm2_b2, ln_w, ln_b,
           gm_w1, gm_b1, gm_w2, gm_b2, hm_w1, hm_b1, hm_w2, hm_b2,
           edge_index, batch):
    N, F = x.shape
    E = edge_index.shape[1]
    S, B, FI = instr_vectors.shape

    NP = (N // 1024 + 1) * 1024
    T = -(-E // (_NW * _CH))
    T = -(-T // 16) * 16
    EP = _NW * T * _CH

    row = edge_index[0]
    col = edge_index[1]
    pad = EP - E
    pad_i = jnp.arange(pad, dtype=jnp.int32)
    rows_p = jnp.concatenate([row, pad_i % N]).reshape(_NW, T, _CH)
    cols_p = jnp.concatenate([col, N + pad_i % (NP - N)]).reshape(_NW, T, _CH)

    ones_ch = jnp.ones((_CH, 128), F32)
    zeros_n = jnp.zeros((NP // _NS, F), F32)

    degparts = _sc_deg_call(cols_p, ones_ch, zeros_n, N, NP, T)

    msg, U1all, OH, cntF = _precompute_call(
        N, B, F, S, x, batch[:, None],
        instr_vectors.reshape(S * B, FI),
        nm1_w1[:F], nm1_b1[None, :], nm1_w2, nm1_b2[None, :],
        nm2_w1[2 * F:],
    )
    deg = _deg_reduce_call(N, NP, degparts)

    step_weights = (
        nm2_w1[:F], nm2_b1[None, :],
        nm2_w1[F:2 * F], nm2_w2, nm2_b2[None, :],
        ln_w[None, :], ln_b[None, :],
        gm_w1, gm_b1[None, :], gm_w2, gm_b2[None, :],
        hm_w1, hm_b1[None, :], hm_w2, hm_b2[None, :],
        nm1_w1[:F], nm1_b1[None, :], nm1_w1[F:], nm1_w2, nm1_b2[None, :],
    )

    gates = []
    hvs = []
    for s in range(S):
        part = _sc_scatter_call(msg, rows_p, cols_p, zeros_n, N, NP, F, T)
        gate, hv_s, msg = _step_call(
            N, NP, B, F, part, x, OH, cntF,
            U1all[s * B:(s + 1) * B], deg, *step_weights)
        gates.append(gate)
        hvs.append(hv_s)

    execution_bitmap = jnp.concatenate(gates, axis=1)
    hv = jnp.stack(hvs, axis=0)
    return (x, execution_bitmap, hv)
